# baseline (speedup 1.0000x reference)
import functools

import jax
import jax.numpy as jnp
from jax import lax
from jax.experimental import pallas as pl
from jax.experimental.pallas import tpu as pltpu
from jax.experimental.pallas import tpu_sc as plsc

_NA = 16604
_NP = 13553
_D = 300
_DP = 320
_K = 16
_B = 4096
_NC, _NS = 2, 16
_NW = _NC * _NS
_G = _B // _NW
_R = 8
_T = 16
_QS = 32
_NCH = _DP // 32


def _tree_sum16_bf16(ref, slot, l, start):
    va, vb = [], []
    for k in range(_K):
        a, b = plsc.unpack(ref[slot, l, k, pl.ds(start, 32)],
                           format=plsc.PackFormat.INTERLEAVED)
        va.append(a)
        vb.append(b)
    while len(va) > 1:
        va = [va[i] + va[i + 1] for i in range(0, len(va), 2)]
        vb = [vb[i] + vb[i + 1] for i in range(0, len(vb), 2)]
    return va[0], vb[0]


def _sc_body(nodes_hbm, a2e_1d, p2e_1d, ap_1d, aa_1d, pa_1d, pp_1d,
             self_hbm, nbr_hbm,
             nodes_v, aidx_v, pidx_v, apx, aax, pax, ppx,
             selfa, selfb, nbrbuf, outbuf,
             semI0, semI1, semI2, semI3, semSelf, semF, rsem):
    cid = lax.axis_index("c")
    sid = lax.axis_index("s")
    wid = sid * _NC + cid
    base = wid * _G

    a2e_hbm = a2e_1d
    p2e_hbm = p2e_1d
    ap_hbm = ap_1d
    aa_hbm = aa_1d
    pa_hbm = pa_1d
    pp_hbm = pp_1d

    pltpu.sync_copy(nodes_hbm.at[pl.ds(base, _G)], nodes_v.at[pl.ds(0, _G)])

    for c in range(_G // 16):
        sl = pl.ds(c * 16, 16)
        nv = nodes_v[sl]
        isa = nv < _NA
        zero = jnp.zeros((16,), jnp.int32)
        aidx_v[sl] = jnp.where(isa, nv, zero)
        pidx_v[sl] = jnp.where(isa, zero, nv - _NA)

    c0 = pltpu.async_copy(ap_hbm.at[aidx_v], apx, semI0)
    c1 = pltpu.async_copy(aa_hbm.at[aidx_v], aax, semI1)
    c2 = pltpu.async_copy(pa_hbm.at[pidx_v], pax, semI2)
    c3 = pltpu.async_copy(pp_hbm.at[pidx_v], ppx, semI3)
    c0.wait(); c1.wait(); c2.wait(); c3.wait()

    def _issue(n, slot):
        node = nodes_v[pl.ds(n, 16)][0]
        isa = node < _NA

        def _starts(t0, x0, t1, x1):
            for hh in range(2):
                hs = pl.ds(hh * (_K // 2), _K // 2)
                pltpu.make_async_copy(t0.at[x0.at[n, hs]],
                                      nbrbuf.at[slot, 0, hs],
                                      rsem.at[slot]).start()
                pltpu.make_async_copy(t1.at[x1.at[n, hs]],
                                      nbrbuf.at[slot, 1, hs],
                                      rsem.at[slot]).start()

        @pl.when(isa)
        def _():
            _starts(p2e_hbm, apx, a2e_hbm, aax)

        @pl.when(jnp.logical_not(isa))
        def _():
            _starts(a2e_hbm, pax, p2e_hbm, ppx)

    for b in range(_R):
        _issue(jnp.int32(b), b)

    for h in range(_G // _QS):
        ca = pltpu.async_copy(a2e_hbm.at[aidx_v.at[pl.ds(h * _QS, _QS)]],
                              selfa, semSelf)
        cb = pltpu.async_copy(p2e_hbm.at[pidx_v.at[pl.ds(h * _QS, _QS)]],
                              selfb, semF)
        ca.wait()
        cb.wait()

        @pl.loop(0, _QS)
        def _(r):
            w16 = jnp.broadcast_to(
                (nodes_v[pl.ds(h * _QS + r, 16)][0] < _NA)
                .astype(jnp.float32), (16,))
            w = plsc.pack(w16, w16, format=plsc.PackFormat.INTERLEAVED)

            @pl.loop(0, _NCH)
            def _(c):
                sl = pl.ds(c * 32, 32)
                selfa[r, sl] = w * selfa[r, sl] + (1.0 - w) * selfb[r, sl]

        pltpu.sync_copy(selfa, self_hbm.at[pl.ds(base + h * _QS, _QS)])

    def _reduce(n, slot, row):
        for l in range(2):
            for hh in range(2):
                hs = pl.ds(hh * (_K // 2), _K // 2)
                pltpu.make_async_copy(a2e_hbm.at[apx.at[n, hs]],
                                      nbrbuf.at[slot, l, hs],
                                      rsem.at[slot]).wait()
        for l in range(2):
            @pl.loop(0, _NCH)
            def _(c):
                start = c * 32
                sa, sb = _tree_sum16_bf16(nbrbuf, slot, l, start)
                outbuf[row, pl.ds(l * _DP + start, 32)] = plsc.pack(
                    sa * (1.0 / _K), sb * (1.0 / _K),
                    format=plsc.PackFormat.INTERLEAVED)

    def _maybe_flush(n):
        @pl.when(lax.rem(n, _T) == _T - 1)
        def _():
            dst = nbr_hbm.at[pl.ds(base + n - (_T - 1), _T)]
            pltpu.make_async_copy(outbuf, dst, semF).start()
            pltpu.make_async_copy(outbuf, dst, semF).wait()

    @pl.loop(0, _G - _R)
    def _(n):
        slot = lax.rem(n, _R)
        _reduce(n, slot, lax.rem(n, _T))
        _issue(n + _R, slot)
        _maybe_flush(n)

    @pl.loop(_G - _R, _G)
    def _(n):
        slot = lax.rem(n, _R)
        _reduce(n, slot, lax.rem(n, _T))
        _maybe_flush(n)


def _sc_gather(nodes, a2e_1d, p2e_1d, ap_1d, aa_1d, pa_1d, pp_1d):
    mesh = plsc.VectorSubcoreMesh(core_axis_name="c", subcore_axis_name="s",
                                  num_cores=_NC, num_subcores=_NS)
    f = pl.kernel(
        _sc_body,
        out_type=(jax.ShapeDtypeStruct((_B, _DP), jnp.bfloat16),
                  jax.ShapeDtypeStruct((_B, 2 * _DP), jnp.bfloat16)),
        mesh=mesh,
        scratch_types=[
            pltpu.VMEM((_G + 16,), jnp.int32),
            pltpu.VMEM((_G,), jnp.int32),
            pltpu.VMEM((_G,), jnp.int32),
            pltpu.VMEM((_G, _K), jnp.int32),
            pltpu.VMEM((_G, _K), jnp.int32),
            pltpu.VMEM((_G, _K), jnp.int32),
            pltpu.VMEM((_G, _K), jnp.int32),
            pltpu.VMEM((_QS, _DP), jnp.bfloat16),
            pltpu.VMEM((_QS, _DP), jnp.bfloat16),
            pltpu.VMEM((_R, 2, _K, _DP), jnp.bfloat16),
            pltpu.VMEM((_T, 2 * _DP), jnp.bfloat16),
        ] + [pltpu.SemaphoreType.DMA] * 6 + [pltpu.SemaphoreType.DMA((_R,))],
        compiler_params=pltpu.CompilerParams(use_tc_tiling_on_sc=False,
                                             needs_layout_passes=False),
        name="hanrec_sc_gather",
    )
    return f(nodes, a2e_1d, p2e_1d, ap_1d, aa_1d, pa_1d, pp_1d)


def _pad_body(x_ref, o_ref):
    o_ref[...] = jnp.pad(x_ref[...], ((0, 0), (0, _DP - _D))).astype(
        jnp.bfloat16)


def _pad_rows(x, rows):
    bm = 1024
    return pl.pallas_call(
        _pad_body,
        grid=(pl.cdiv(rows, bm),),
        in_specs=[pl.BlockSpec((bm, _D), lambda i: (i, 0))],
        out_specs=pl.BlockSpec((bm, _DP), lambda i: (i, 0)),
        out_shape=jax.ShapeDtypeStruct((rows, _DP), jnp.bfloat16),
        name="hanrec_pad",
    )(x)


def _mm_body(self_ref, nbr_ref, wa_ref, wb_ref, wc_ref, b1_ref, out_ref):
    acc = jnp.dot(self_ref[...], wa_ref[...], preferred_element_type=jnp.float32)
    acc += jnp.dot(nbr_ref[:, :_DP], wb_ref[...],
                   preferred_element_type=jnp.float32)
    acc += jnp.dot(nbr_ref[:, _DP:], wc_ref[...],
                   preferred_element_type=jnp.float32)
    out_ref[...] = acc + b1_ref[...]


def _tc_linear(self_o, nbr_o, wa, wb, wc, b1):
    bm = 512
    return pl.pallas_call(
        _mm_body,
        grid=(_B // bm,),
        in_specs=[
            pl.BlockSpec((bm, _DP), lambda i: (i, 0)),
            pl.BlockSpec((bm, 2 * _DP), lambda i: (i, 0)),
            pl.BlockSpec((_DP, _D), lambda i: (0, 0)),
            pl.BlockSpec((_DP, _D), lambda i: (0, 0)),
            pl.BlockSpec((_DP, _D), lambda i: (0, 0)),
            pl.BlockSpec((1, _D), lambda i: (0, 0)),
        ],
        out_specs=pl.BlockSpec((bm, _D), lambda i: (i, 0)),
        out_shape=jax.ShapeDtypeStruct((_B, _D), jnp.float32),
        name="hanrec_linear",
    )(self_o, nbr_o, wa, wb, wc, b1)


def kernel(nodes, a2e, p2e, ap_neighbors, aa_neighbors, pa_neighbors,
           pp_neighbors, W1, b1):
    nodes_i = nodes.astype(jnp.int32)
    ap = ap_neighbors.astype(jnp.int32)
    aa = aa_neighbors.astype(jnp.int32)
    pa = pa_neighbors.astype(jnp.int32)
    pp = pp_neighbors.astype(jnp.int32)
    a2e_p = _pad_rows(a2e, _NA)
    p2e_p = _pad_rows(p2e, _NP)
    wa = jnp.pad(W1[0:_D], ((0, _DP - _D), (0, 0)))
    wb = jnp.pad(W1[_D:2 * _D], ((0, _DP - _D), (0, 0)))
    wc = jnp.pad(W1[2 * _D:3 * _D], ((0, _DP - _D), (0, 0)))
    self_o, nbr_o = _sc_gather(nodes_i, a2e_p, p2e_p, ap, aa, pa, pp)
    return _tc_linear(self_o, nbr_o, wa, wb, wc, b1.reshape(1, _D))


# speedup vs baseline: 1.0768x; 1.0768x over previous
import functools

import jax
import jax.numpy as jnp
from jax import lax
from jax.experimental import pallas as pl
from jax.experimental.pallas import tpu as pltpu
from jax.experimental.pallas import tpu_sc as plsc

_NA = 16604
_NP = 13553
_D = 300
_DP = 304
_K = 16
_B = 4096
_NC, _NS = 2, 16
_NW = _NC * _NS
_G = _B // _NW
_R = 4
_T = 32
_NCH = _DP // 16


def _tree_sum16(ref, slot, l, start):
    vs = [ref[slot, l, k, pl.ds(start, 16)] for k in range(_K)]
    while len(vs) > 1:
        vs = [vs[i] + vs[i + 1] for i in range(0, len(vs), 2)]
    return vs[0]


def _sc_body(nodes_hbm, a2e_1d, p2e_1d, ap_1d, aa_1d, pa_1d, pp_1d,
             self_hbm, nbr_hbm,
             nodes_v, aidx_v, pidx_v, apx, aax, pax, ppx,
             selfa, selfb, nbrbuf, outbuf,
             semI0, semI1, semI2, semI3, semSelf, semF,
             sem00, sem01, sem10, sem11, sem20, sem21, sem30, sem31):
    cid = lax.axis_index("c")
    sid = lax.axis_index("s")
    wid = sid * _NC + cid
    base = wid * _G
    ring = ((sem00, sem01), (sem10, sem11), (sem20, sem21), (sem30, sem31))

    a2e_hbm = a2e_1d
    p2e_hbm = p2e_1d
    ap_hbm = ap_1d
    aa_hbm = aa_1d
    pa_hbm = pa_1d
    pp_hbm = pp_1d

    pltpu.sync_copy(nodes_hbm.at[pl.ds(base, _G)], nodes_v.at[pl.ds(0, _G)])

    for c in range(_G // 16):
        sl = pl.ds(c * 16, 16)
        nv = nodes_v[sl]
        isa = nv < _NA
        zero = jnp.zeros((16,), jnp.int32)
        aidx_v[sl] = jnp.where(isa, nv, zero)
        pidx_v[sl] = jnp.where(isa, zero, nv - _NA)

    c0 = pltpu.async_copy(ap_hbm.at[aidx_v], apx, semI0)
    c1 = pltpu.async_copy(aa_hbm.at[aidx_v], aax, semI1)
    c2 = pltpu.async_copy(pa_hbm.at[pidx_v], pax, semI2)
    c3 = pltpu.async_copy(pp_hbm.at[pidx_v], ppx, semI3)
    c0.wait(); c1.wait(); c2.wait(); c3.wait()

    for h in range(2):
        ca = pltpu.async_copy(a2e_hbm.at[aidx_v.at[pl.ds(h * 64, 64)]],
                              selfa, semSelf)
        cb = pltpu.async_copy(p2e_hbm.at[pidx_v.at[pl.ds(h * 64, 64)]],
                              selfb, semF)
        ca.wait()
        cb.wait()

        @pl.loop(0, 64)
        def _(r):
            w = jnp.broadcast_to(
                (nodes_v[pl.ds(h * 64 + r, 16)][0] < _NA).astype(jnp.float32),
                (16,))

            @pl.loop(0, _NCH)
            def _(c):
                sl = pl.ds(c * 16, 16)
                selfa[r, sl] = w * selfa[r, sl] + (1.0 - w) * selfb[r, sl]

        pltpu.sync_copy(selfa, self_hbm.at[pl.ds(base + h * 64, 64)])

    def _issue(n, slot):
        node = nodes_v[pl.ds(n, 16)][0]
        isa = node < _NA

        @pl.when(isa)
        def _():
            pltpu.make_async_copy(p2e_hbm.at[apx.at[n]], nbrbuf.at[slot, 0],
                                  ring[slot][0]).start()
            pltpu.make_async_copy(a2e_hbm.at[aax.at[n]], nbrbuf.at[slot, 1],
                                  ring[slot][1]).start()

        @pl.when(jnp.logical_not(isa))
        def _():
            pltpu.make_async_copy(a2e_hbm.at[pax.at[n]], nbrbuf.at[slot, 0],
                                  ring[slot][0]).start()
            pltpu.make_async_copy(p2e_hbm.at[ppx.at[n]], nbrbuf.at[slot, 1],
                                  ring[slot][1]).start()

    def _reduce(n, slot, row):
        for l in range(2):
            pltpu.make_async_copy(a2e_hbm.at[apx.at[n]], nbrbuf.at[slot, l],
                                  ring[slot][l]).wait()
        for l in range(2):
            @pl.loop(0, _NCH)
            def _(c):
                start = c * 16
                s = _tree_sum16(nbrbuf, slot, l, start)
                outbuf[row, pl.ds(l * _DP + start, 16)] = s * (1.0 / _K)

    for b in range(_R):
        _issue(jnp.int32(b), b)

    n_tiles = _G // _T
    for t in range(n_tiles):
        i0, i1 = t * (_T // _R), (t + 1) * (_T // _R)
        last = t == n_tiles - 1

        @pl.loop(i0, i1 - (1 if last else 0))
        def _(i):
            n = i * _R
            for b in range(_R):
                _reduce(n + b, b, n + b - t * _T)
                _issue(n + b + _R, b)

        if last:
            for b in range(_R):
                n = jnp.int32(_G - _R + b)
                _reduce(n, b, n - t * _T)
        pltpu.make_async_copy(outbuf, nbr_hbm.at[pl.ds(base + t * _T, _T)],
                              semF).start()
        pltpu.make_async_copy(outbuf, nbr_hbm.at[pl.ds(base + t * _T, _T)],
                              semF).wait()


def _sc_gather(nodes, a2e_1d, p2e_1d, ap_1d, aa_1d, pa_1d, pp_1d):
    mesh = plsc.VectorSubcoreMesh(core_axis_name="c", subcore_axis_name="s",
                                  num_cores=_NC, num_subcores=_NS)
    f = pl.kernel(
        _sc_body,
        out_type=(jax.ShapeDtypeStruct((_B, _DP), jnp.float32),
                  jax.ShapeDtypeStruct((_B, 2 * _DP), jnp.float32)),
        mesh=mesh,
        scratch_types=[
            pltpu.VMEM((_G + 16,), jnp.int32),
            pltpu.VMEM((_G,), jnp.int32),
            pltpu.VMEM((_G,), jnp.int32),
            pltpu.VMEM((_G, _K), jnp.int32),
            pltpu.VMEM((_G, _K), jnp.int32),
            pltpu.VMEM((_G, _K), jnp.int32),
            pltpu.VMEM((_G, _K), jnp.int32),
            pltpu.VMEM((_G // 2, _DP), jnp.float32),
            pltpu.VMEM((_G // 2, _DP), jnp.float32),
            pltpu.VMEM((_R, 2, _K, _DP), jnp.float32),
            pltpu.VMEM((_T, 2 * _DP), jnp.float32),
        ] + [pltpu.SemaphoreType.DMA] * 14,
        compiler_params=pltpu.CompilerParams(use_tc_tiling_on_sc=False),
        name="hanrec_sc_gather",
    )
    return f(nodes, a2e_1d, p2e_1d, ap_1d, aa_1d, pa_1d, pp_1d)


def _pad_body(x_ref, o_ref):
    o_ref[...] = jnp.pad(x_ref[...], ((0, 0), (0, _DP - _D)))


def _pad_rows(x, rows):
    bm = 1024
    return pl.pallas_call(
        _pad_body,
        grid=(pl.cdiv(rows, bm),),
        in_specs=[pl.BlockSpec((bm, _D), lambda i: (i, 0))],
        out_specs=pl.BlockSpec((bm, _DP), lambda i: (i, 0)),
        out_shape=jax.ShapeDtypeStruct((rows, _DP), jnp.float32),
        name="hanrec_pad",
    )(x)


def _mm_body(self_ref, nbr_ref, wa_ref, wb_ref, wc_ref, b1_ref, out_ref):
    acc = jnp.dot(self_ref[...], wa_ref[...], preferred_element_type=jnp.float32)
    acc += jnp.dot(nbr_ref[:, :_DP], wb_ref[...],
                   preferred_element_type=jnp.float32)
    acc += jnp.dot(nbr_ref[:, _DP:], wc_ref[...],
                   preferred_element_type=jnp.float32)
    out_ref[...] = acc + b1_ref[...]


def _tc_linear(self_o, nbr_o, wa, wb, wc, b1):
    bm = 512
    return pl.pallas_call(
        _mm_body,
        grid=(_B // bm,),
        in_specs=[
            pl.BlockSpec((bm, _DP), lambda i: (i, 0)),
            pl.BlockSpec((bm, 2 * _DP), lambda i: (i, 0)),
            pl.BlockSpec((_DP, _D), lambda i: (0, 0)),
            pl.BlockSpec((_DP, _D), lambda i: (0, 0)),
            pl.BlockSpec((_DP, _D), lambda i: (0, 0)),
            pl.BlockSpec((1, _D), lambda i: (0, 0)),
        ],
        out_specs=pl.BlockSpec((bm, _D), lambda i: (i, 0)),
        out_shape=jax.ShapeDtypeStruct((_B, _D), jnp.float32),
        name="hanrec_linear",
    )(self_o, nbr_o, wa, wb, wc, b1)


def kernel(nodes, a2e, p2e, ap_neighbors, aa_neighbors, pa_neighbors,
           pp_neighbors, W1, b1):
    nodes_i = nodes.astype(jnp.int32)
    ap = ap_neighbors.astype(jnp.int32)
    aa = aa_neighbors.astype(jnp.int32)
    pa = pa_neighbors.astype(jnp.int32)
    pp = pp_neighbors.astype(jnp.int32)
    a2e_p = _pad_rows(a2e, _NA)
    p2e_p = _pad_rows(p2e, _NP)
    wa = jnp.pad(W1[0:_D], ((0, _DP - _D), (0, 0)))
    wb = jnp.pad(W1[_D:2 * _D], ((0, _DP - _D), (0, 0)))
    wc = jnp.pad(W1[2 * _D:3 * _D], ((0, _DP - _D), (0, 0)))
    self_o, nbr_o = _sc_gather(nodes_i, a2e_p, p2e_p, ap, aa, pa, pp)
    return _tc_linear(self_o, nbr_o, wa, wb, wc, b1.reshape(1, _D))
